# baseline (speedup 1.0000x reference)
import functools

import jax
import jax.numpy as jnp
import numpy as np
from jax import lax
from jax.experimental import pallas as pl
from jax.experimental.pallas import tpu as pltpu
from jax.experimental.pallas import tpu_sc as plsc

D = 128
_NC, _NS = 2, 16
_NW = _NC * _NS
_CHUNK = 64
_NBUF = 4
_DP = D // 2


def _proj_body(x_ref, wt_ref, b_ref, o_ref):
    o_ref[...] = lax.dot_general(
        x_ref[...], wt_ref[...], (((1,), (1,)), ((), ())),
        preferred_element_type=jnp.float32,
    ) + b_ref[...]


def _project(table, wt, bias, row_block):
    rows = table.shape[0]
    grid = rows // row_block
    return pl.pallas_call(
        _proj_body,
        grid=(grid,),
        in_specs=[
            pl.BlockSpec((row_block, D), lambda i: (i, 0)),
            pl.BlockSpec((D, D), lambda i: (0, 0)),
            pl.BlockSpec((1, D), lambda i: (0, 0)),
        ],
        out_specs=pl.BlockSpec((row_block, D), lambda i: (i, 0)),
        out_shape=jax.ShapeDtypeStruct((rows, D), jnp.float32),
    )(table, wt, bias)


def _gather_add_body(widx_hbm, ridx_hbm, wtab_hbm, rtab_hbm, out_hbm,
                     widx_all, ridx_all, rows_w, rows_r, rows_o,
                     sem_g0, sem_g1, sem_g2, sem_g3,
                     sem_wb0, sem_wb1, sem_wb2, sem_wb3):
    n = out_hbm.shape[0]
    rpw = n // _NW
    nchunk = rpw // _CHUNK
    wid = lax.axis_index("s") * _NC + lax.axis_index("c")
    base0 = wid * rpw
    sem_g = (sem_g0, sem_g1, sem_g2, sem_g3)
    sem_wb = (sem_wb0, sem_wb1, sem_wb2, sem_wb3)

    pltpu.sync_copy(widx_hbm.at[pl.ds(base0, rpw)], widx_all)
    pltpu.sync_copy(ridx_hbm.at[pl.ds(base0, rpw)], ridx_all)

    def _issue_gathers(ci, b):
        pltpu.async_copy(wtab_hbm.at[widx_all.at[pl.ds(ci * _CHUNK, _CHUNK)]],
                         rows_w.at[b], sem_g[b])
        pltpu.async_copy(rtab_hbm.at[ridx_all.at[pl.ds(ci * _CHUNK, _CHUNK)]],
                         rows_r.at[b], sem_g[b])

    def _wait_gathers(ci, b):
        pltpu.make_async_copy(
            wtab_hbm.at[widx_all.at[pl.ds(ci * _CHUNK, _CHUNK)]],
            rows_w.at[b], sem_g[b]).wait()
        pltpu.make_async_copy(
            rtab_hbm.at[ridx_all.at[pl.ds(ci * _CHUNK, _CHUNK)]],
            rows_r.at[b], sem_g[b]).wait()

    for b in range(_NBUF):
        _issue_gathers(b, b)

    @pl.loop(0, nchunk, step=_NBUF)
    def _sweep(ci0):
        for b in range(_NBUF):
            ci = ci0 + b
            base = base0 + ci * _CHUNK
            _wait_gathers(ci, b)

            @pl.when(ci >= _NBUF)
            def _():
                pltpu.make_async_copy(
                    rows_o.at[b],
                    out_hbm.at[pl.ds(base - _NBUF * _CHUNK, _CHUNK)],
                    sem_wb[b]).wait()

            @pl.loop(0, _CHUNK)
            def _row(r):
                for s in range(D // 32):
                    sl = pl.ds(s * 16, 16)
                    wlo, whi = plsc.unpack(
                        plsc.bitcast(rows_w[b, r, sl], jnp.bfloat16),
                        format=plsc.PackFormat.INTERLEAVED,
                        preferred_element_type=jnp.float32)
                    rlo, rhi = plsc.unpack(
                        plsc.bitcast(rows_r[b, r, sl], jnp.bfloat16),
                        format=plsc.PackFormat.INTERLEAVED,
                        preferred_element_type=jnp.float32)
                    rows_o[b, r, pl.ds(s * 32, 16)] = wlo + rlo
                    rows_o[b, r, pl.ds(s * 32 + 16, 16)] = whi + rhi

            pltpu.async_copy(rows_o.at[b], out_hbm.at[pl.ds(base, _CHUNK)],
                             sem_wb[b])

            nxt = ci + _NBUF
            @pl.when(nxt < nchunk)
            def _():
                _issue_gathers(nxt, b)

    for b in range(_NBUF):
        last = nchunk - _NBUF + b
        pltpu.make_async_copy(
            rows_o.at[b], out_hbm.at[pl.ds(base0 + last * _CHUNK, _CHUNK)],
            sem_wb[b]).wait()


def _gather_add(widx, ridx, wtab, rtab):
    n = widx.shape[0]
    rpw = n // _NW
    mesh = plsc.VectorSubcoreMesh(core_axis_name="c", subcore_axis_name="s")
    fn = pl.kernel(
        _gather_add_body,
        out_type=jax.ShapeDtypeStruct((n, D), jnp.float32),
        mesh=mesh,
        compiler_params=pltpu.CompilerParams(needs_layout_passes=False,
                                             use_tc_tiling_on_sc=False),
        scratch_types=[
            pltpu.VMEM((rpw,), jnp.int32),
            pltpu.VMEM((rpw,), jnp.int32),
            pltpu.VMEM((_NBUF, _CHUNK, _DP), jnp.int32),
            pltpu.VMEM((_NBUF, _CHUNK, _DP), jnp.int32),
            pltpu.VMEM((_NBUF, _CHUNK, D), jnp.float32),
            pltpu.SemaphoreType.DMA,
            pltpu.SemaphoreType.DMA,
            pltpu.SemaphoreType.DMA,
            pltpu.SemaphoreType.DMA,
            pltpu.SemaphoreType.DMA,
            pltpu.SemaphoreType.DMA,
            pltpu.SemaphoreType.DMA,
            pltpu.SemaphoreType.DMA,
        ],
    )
    return fn(widx, ridx, wtab, rtab)


_J = np.arange(D // 2)
_PERM = np.concatenate([32 * (_J // 16) + _J % 16,
                        32 * (_J // 16) + 16 + _J % 16])


def _pack_rows(tab_f32):
    lo = lax.convert_element_type(
        lax.bitcast_convert_type(tab_f32[:, :_DP].astype(jnp.bfloat16),
                                 jnp.uint16), jnp.uint32)
    hi = lax.convert_element_type(
        lax.bitcast_convert_type(tab_f32[:, _DP:].astype(jnp.bfloat16),
                                 jnp.uint16), jnp.uint32)
    return lax.bitcast_convert_type((hi << 16) | lo, jnp.int32)


@jax.jit
def kernel(word, remainder, word_table, rem_table, W, b):
    bsz, seq = word.shape
    Wp = W[_PERM, :]
    bp = b[_PERM]
    word_proj = _project(word_table, Wp[:, :D],
                         jnp.zeros((1, D), jnp.float32), row_block=2000)
    rem_proj = _project(rem_table, Wp[:, D:], bp.reshape(1, D), row_block=512)
    out = _gather_add(word.reshape(-1), remainder.reshape(-1),
                      _pack_rows(word_proj), _pack_rows(rem_proj))
    return out.reshape(bsz, seq, D)


# speedup vs baseline: 1.1145x; 1.1145x over previous
import functools

import jax
import jax.numpy as jnp
from jax import lax
from jax.experimental import pallas as pl
from jax.experimental.pallas import tpu as pltpu
from jax.experimental.pallas import tpu_sc as plsc

D = 128
_NC, _NS = 2, 16
_NW = _NC * _NS
_CHUNK = 80
_NBUF = 2


def _proj_body(x_ref, wt_ref, b_ref, o_ref):
    o_ref[...] = lax.dot_general(
        x_ref[...], wt_ref[...], (((1,), (1,)), ((), ())),
        preferred_element_type=jnp.float32,
    ) + b_ref[...]


def _project(table, wt, bias, row_block):
    rows = table.shape[0]
    grid = rows // row_block
    return pl.pallas_call(
        _proj_body,
        grid=(grid,),
        in_specs=[
            pl.BlockSpec((row_block, D), lambda i: (i, 0)),
            pl.BlockSpec((D, D), lambda i: (0, 0)),
            pl.BlockSpec((1, D), lambda i: (0, 0)),
        ],
        out_specs=pl.BlockSpec((row_block, D), lambda i: (i, 0)),
        out_shape=jax.ShapeDtypeStruct((rows, D), jnp.float32),
    )(table, wt, bias)


def _gather_add_body(widx_hbm, ridx_hbm, wtab_hbm, rtab_hbm, out_hbm,
                     widx_all, ridx_all, rows_w, rows_r, rows_o,
                     sem_g0, sem_g1, sem_wb0, sem_wb1):
    n = out_hbm.shape[0]
    rpw = n // _NW
    nchunk = rpw // _CHUNK
    wid = lax.axis_index("s") * _NC + lax.axis_index("c")
    base0 = wid * rpw
    sem_g = (sem_g0, sem_g1)
    sem_wb = (sem_wb0, sem_wb1)

    pltpu.sync_copy(widx_hbm.at[pl.ds(base0, rpw)], widx_all)
    pltpu.sync_copy(ridx_hbm.at[pl.ds(base0, rpw)], ridx_all)

    def _issue_gathers(ci, b):
        pltpu.async_copy(wtab_hbm.at[widx_all.at[pl.ds(ci * _CHUNK, _CHUNK)]],
                         rows_w.at[b], sem_g[b])
        pltpu.async_copy(rtab_hbm.at[ridx_all.at[pl.ds(ci * _CHUNK, _CHUNK)]],
                         rows_r.at[b], sem_g[b])

    def _wait_gathers(ci, b):
        pltpu.make_async_copy(
            wtab_hbm.at[widx_all.at[pl.ds(ci * _CHUNK, _CHUNK)]],
            rows_w.at[b], sem_g[b]).wait()
        pltpu.make_async_copy(
            rtab_hbm.at[ridx_all.at[pl.ds(ci * _CHUNK, _CHUNK)]],
            rows_r.at[b], sem_g[b]).wait()

    for b in range(_NBUF):
        _issue_gathers(b, b)

    @pl.loop(0, nchunk, step=_NBUF)
    def _sweep(ci0):
        for b in range(_NBUF):
            ci = ci0 + b
            base = base0 + ci * _CHUNK
            _wait_gathers(ci, b)

            @pl.when(ci >= _NBUF)
            def _():
                pltpu.make_async_copy(
                    rows_o.at[b],
                    out_hbm.at[pl.ds(base - _NBUF * _CHUNK, _CHUNK)],
                    sem_wb[b]).wait()

            @pl.loop(0, _CHUNK)
            def _row(r):
                for g in range(D // 16):
                    sl = pl.ds(g * 16, 16)
                    rows_o[b, r, sl] = rows_w[b, r, sl] + rows_r[b, r, sl]

            pltpu.async_copy(rows_o.at[b], out_hbm.at[pl.ds(base, _CHUNK)],
                             sem_wb[b])

            nxt = ci + _NBUF
            @pl.when(nxt < nchunk)
            def _():
                _issue_gathers(nxt, b)

    for b in range(_NBUF):
        last = nchunk - _NBUF + b
        pltpu.make_async_copy(
            rows_o.at[b], out_hbm.at[pl.ds(base0 + last * _CHUNK, _CHUNK)],
            sem_wb[b]).wait()


def _gather_add(widx, ridx, wtab, rtab):
    n = widx.shape[0]
    rpw = n // _NW
    mesh = plsc.VectorSubcoreMesh(core_axis_name="c", subcore_axis_name="s")
    fn = pl.kernel(
        _gather_add_body,
        out_type=jax.ShapeDtypeStruct((n, D), jnp.float32),
        mesh=mesh,
        scratch_types=[
            pltpu.VMEM((rpw,), jnp.int32),
            pltpu.VMEM((rpw,), jnp.int32),
            pltpu.VMEM((_NBUF, _CHUNK, D), jnp.float32),
            pltpu.VMEM((_NBUF, _CHUNK, D), jnp.float32),
            pltpu.VMEM((_NBUF, _CHUNK, D), jnp.float32),
            pltpu.SemaphoreType.DMA,
            pltpu.SemaphoreType.DMA,
            pltpu.SemaphoreType.DMA,
            pltpu.SemaphoreType.DMA,
        ],
    )
    return fn(widx, ridx, wtab, rtab)


@jax.jit
def kernel(word, remainder, word_table, rem_table, W, b):
    bsz, seq = word.shape
    word_proj = _project(word_table, W[:, :D], jnp.zeros((1, D), jnp.float32),
                         row_block=2000)
    rem_proj = _project(rem_table, W[:, D:], b.reshape(1, D), row_block=512)
    out = _gather_add(word.reshape(-1), remainder.reshape(-1),
                      word_proj, rem_proj)
    return out.reshape(bsz, seq, D)
